# baseline (speedup 1.0000x reference)
import functools

import jax
import jax.numpy as jnp
from jax import lax
from jax.experimental import pallas as pl
from jax.experimental.pallas import tpu as pltpu
from jax.experimental.pallas import tpu_sc as plsc

N = 10000
D = 128
E = 320000
NC = 2
NS = 16
NW = NC * NS
FPW = D // NW
CH = 128
CPW = 80
EP = NW * CPW * CH
EBP = EP // CH
CBR = 64
NCH = EBP // CBR
NP = 10240
BLK = 512

_mesh = plsc.VectorSubcoreMesh(core_axis_name="c", subcore_axis_name="s")
_sc_params = pltpu.CompilerParams(
    needs_layout_passes=False, use_tc_tiling_on_sc=False)


@functools.partial(
    pl.kernel,
    out_type=jax.ShapeDtypeStruct((NW, NP), jnp.float32),
    mesh=_mesh,
    compiler_params=_sc_params,
    scratch_types=[
        pltpu.VMEM((CPW, CH), jnp.int32),
        pltpu.VMEM((NP,), jnp.float32),
    ],
)
def _sc_degrees(dst_hbm, zer_hbm, deg_hbm, idx_v, acc_v):
    cid = lax.axis_index("c")
    sid = lax.axis_index("s")
    wid = sid * NC + cid
    pltpu.sync_copy(zer_hbm, acc_v)
    pltpu.sync_copy(dst_hbm.at[wid], idx_v)

    def row(j, carry):
        ones = jnp.ones((16,), jnp.float32)
        for k in range(CH // 16):
            d16 = idx_v[j, pl.ds(k * 16, 16)]
            plsc.addupdate_scatter(acc_v, [d16], ones)
        return carry

    lax.fori_loop(0, CPW, row, 0)
    pltpu.sync_copy(acc_v, deg_hbm.at[wid])


@functools.partial(
    pl.kernel,
    out_type=jax.ShapeDtypeStruct((D, NP), jnp.float32),
    mesh=_mesh,
    compiler_params=_sc_params,
    scratch_types=[
        [pltpu.VMEM((NP,), jnp.float32) for _ in range(FPW)],
        [pltpu.VMEM((NP,), jnp.float32) for _ in range(FPW)],
        [pltpu.VMEM((CBR, CH), jnp.int32) for _ in range(2)],
        pltpu.SemaphoreType.DMA,
        pltpu.SemaphoreType.DMA,
    ],
)
def _sc_scatter(yt_hbm, pk_hbm, zer_hbm, out_hbm, ys, ac, pbuf, semA, semB):
    cid = lax.axis_index("c")
    sid = lax.axis_index("s")
    wid = sid * NC + cid
    f0 = wid * FPW
    for f in range(FPW):
        pltpu.sync_copy(yt_hbm.at[f0 + f], ys[f])
        pltpu.sync_copy(zer_hbm, ac[f])

    def start(c, b, sem):
        pltpu.async_copy(pk_hbm.at[pl.ds(c * CBR, CBR)], pbuf[b], sem)

    def drain(b, sem):
        pltpu.make_async_copy(pk_hbm.at[pl.ds(0, CBR)], pbuf[b], sem).wait()

    def process(b):
        def row(r, carry2):
            for h in range(2):
                ks = range(h * 4, h * 4 + 4)
                p = [pbuf[b][r, pl.ds(k * 16, 16)] for k in ks]
                s = [p[k] & 0xFFFF for k in range(4)]
                d = [p[k] >> 16 for k in range(4)]
                g = [[plsc.load_gather(ys[f], [s[k]]) for f in range(FPW)]
                     for k in range(4)]
                for k in range(4):
                    for f in range(FPW):
                        plsc.addupdate_scatter(ac[f], [d[k]], g[k][f])
            return carry2

        lax.fori_loop(0, CBR, row, 0)

    start(0, 0, semA)

    def pair(i, carry):
        c0 = 2 * i
        drain(0, semA)
        start(c0 + 1, 1, semB)
        process(0)
        drain(1, semB)

        @pl.when(c0 + 2 < NCH)
        def _():
            start(c0 + 2, 0, semA)

        process(1)
        return carry

    lax.fori_loop(0, NCH // 2, pair, 0)
    for f in range(FPW):
        pltpu.sync_copy(ac[f], out_hbm.at[f0 + f])


def _prep_body(x_ref, w_ref, degp_ref, yt_ref, dinv_ref):
    deg = jnp.sum(degp_ref[...], axis=0, keepdims=True) + 1.0
    dinv = lax.rsqrt(deg)
    h = jnp.dot(x_ref[...], w_ref[...], preferred_element_type=jnp.float32)
    yt_ref[...] = h.T * dinv
    dinv_ref[...] = jnp.broadcast_to(dinv, (8, BLK))


def _tc_prep(xp, w, degp):
    return pl.pallas_call(
        _prep_body,
        grid=(NP // BLK,),
        in_specs=[
            pl.BlockSpec((BLK, D), lambda i: (i, 0)),
            pl.BlockSpec((D, D), lambda i: (0, 0)),
            pl.BlockSpec((NW, BLK), lambda i: (0, i)),
        ],
        out_specs=[
            pl.BlockSpec((D, BLK), lambda i: (0, i)),
            pl.BlockSpec((8, BLK), lambda i: (0, i)),
        ],
        out_shape=[
            jax.ShapeDtypeStruct((D, NP), jnp.float32),
            jax.ShapeDtypeStruct((8, NP), jnp.float32),
        ],
    )(xp, w, degp)


def _mid_body(p_ref, y0_ref, dinv_ref, b_ref, w_ref, y1_ref):
    dinv = dinv_ref[0:1, :]
    agg = p_ref[...] + y0_ref[...]
    h = jnp.maximum(agg * dinv + b_ref[...], 0.0)
    y1 = lax.dot_general(w_ref[...], h, (((0,), (0,)), ((), ())),
                         preferred_element_type=jnp.float32)
    y1_ref[...] = y1 * dinv


def _tc_mid(p, y0t, dinv8, bc, w):
    return pl.pallas_call(
        _mid_body,
        grid=(NP // BLK,),
        in_specs=[
            pl.BlockSpec((D, BLK), lambda i: (0, i)),
            pl.BlockSpec((D, BLK), lambda i: (0, i)),
            pl.BlockSpec((8, BLK), lambda i: (0, i)),
            pl.BlockSpec((D, 1), lambda i: (0, 0)),
            pl.BlockSpec((D, D), lambda i: (0, 0)),
        ],
        out_specs=pl.BlockSpec((D, BLK), lambda i: (0, i)),
        out_shape=jax.ShapeDtypeStruct((D, NP), jnp.float32),
    )(p, y0t, dinv8, bc, w)


def _fin_body(q_ref, y1_ref, dinv_ref, b_ref, z_ref):
    dinv = dinv_ref[0:1, :]
    zt = (q_ref[...] + y1_ref[...]) * dinv + b_ref[...]
    z_ref[...] = zt.T


def _tc_fin(q, y1t, dinv8, bc):
    return pl.pallas_call(
        _fin_body,
        grid=(NP // BLK,),
        in_specs=[
            pl.BlockSpec((D, BLK), lambda i: (0, i)),
            pl.BlockSpec((D, BLK), lambda i: (0, i)),
            pl.BlockSpec((8, BLK), lambda i: (0, i)),
            pl.BlockSpec((D, 1), lambda i: (0, 0)),
        ],
        out_specs=pl.BlockSpec((BLK, D), lambda i: (i, 0)),
        out_shape=jax.ShapeDtypeStruct((NP, D), jnp.float32),
    )(q, y1t, dinv8, bc)


def kernel(x1, edge_index1, x2, edge_index2, W0, b0, W1, b1):
    zer = jnp.zeros((NP,), jnp.float32)
    b0c = b0.reshape(D, 1)
    b1c = b1.reshape(D, 1)
    x1p = jnp.pad(x1, ((0, NP - N), (0, 0)))
    x2p = jnp.pad(x2, ((0, NP - N), (0, 0)))

    def prep_edges(ei):
        pad = jnp.full((EP - E,), N, dtype=jnp.int32)
        srcf = jnp.concatenate([ei[0].astype(jnp.int32), pad])
        dstf = jnp.concatenate([ei[1].astype(jnp.int32), pad])
        pk = (srcf | (dstf << 16)).reshape(EBP, CH)
        return pk, dstf.reshape(NW, CPW, CH)

    pk1, dstp1 = prep_edges(edge_index1)
    pk2, dstp2 = prep_edges(edge_index2)

    degp1 = _sc_degrees(dstp1, zer)
    degp2 = _sc_degrees(dstp2, zer)

    def backbone(xp, degp, pk):
        y0t, dinv8 = _tc_prep(xp, W0, degp)
        p = _sc_scatter(y0t, pk, zer)
        y1t = _tc_mid(p, y0t, dinv8, b0c, W1)
        q = _sc_scatter(y1t, pk, zer)
        return _tc_fin(q, y1t, dinv8, b1c)

    z1 = backbone(x1p, degp1, pk1)[:N]
    z2 = backbone(x2p, degp2, pk2)[:N]
    return (z1, z2)


# speedup vs baseline: 1.0361x; 1.0361x over previous
import functools

import jax
import jax.numpy as jnp
from jax import lax
from jax.experimental import pallas as pl
from jax.experimental.pallas import tpu as pltpu
from jax.experimental.pallas import tpu_sc as plsc

N = 10000
D = 128
E = 320000
NC = 2
NS = 16
NW = NC * NS
FPW = D // NW
CH = 128
CPW = 80
EP = NW * CPW * CH
EBP = EP // CH
CBR = 64
NCH = EBP // CBR
NP = 10240
BLK = 512

_mesh = plsc.VectorSubcoreMesh(core_axis_name="c", subcore_axis_name="s")
_sc_params = pltpu.CompilerParams(
    needs_layout_passes=False, use_tc_tiling_on_sc=False)


@functools.partial(
    pl.kernel,
    out_type=jax.ShapeDtypeStruct((NW, NP), jnp.float32),
    mesh=_mesh,
    compiler_params=_sc_params,
    scratch_types=[
        pltpu.VMEM((CPW, CH), jnp.int32),
        pltpu.VMEM((NP,), jnp.float32),
    ],
)
def _sc_degrees(dst_hbm, zer_hbm, deg_hbm, idx_v, acc_v):
    cid = lax.axis_index("c")
    sid = lax.axis_index("s")
    wid = sid * NC + cid
    pltpu.sync_copy(zer_hbm, acc_v)
    pltpu.sync_copy(dst_hbm.at[wid], idx_v)

    def row(j, carry):
        ones = jnp.ones((16,), jnp.float32)
        for k in range(CH // 16):
            d16 = idx_v[j, pl.ds(k * 16, 16)]
            plsc.addupdate_scatter(acc_v, [d16], ones)
        return carry

    lax.fori_loop(0, CPW, row, 0)
    pltpu.sync_copy(acc_v, deg_hbm.at[wid])


@functools.partial(
    pl.kernel,
    out_type=jax.ShapeDtypeStruct((D, NP), jnp.float32),
    mesh=_mesh,
    compiler_params=_sc_params,
    scratch_types=[
        [pltpu.VMEM((NP,), jnp.float32) for _ in range(FPW)],
        [pltpu.VMEM((NP,), jnp.float32) for _ in range(FPW)],
        [pltpu.VMEM((CBR, CH), jnp.int32) for _ in range(2)],
        [pltpu.VMEM((CBR, CH), jnp.int32) for _ in range(2)],
        pltpu.SemaphoreType.DMA,
        pltpu.SemaphoreType.DMA,
    ],
)
def _sc_scatter(yt_hbm, src_hbm, dst_hbm, zer_hbm, out_hbm,
                ys, ac, sbuf, dbuf, semA, semB):
    cid = lax.axis_index("c")
    sid = lax.axis_index("s")
    wid = sid * NC + cid
    f0 = wid * FPW
    for f in range(FPW):
        pltpu.sync_copy(yt_hbm.at[f0 + f], ys[f])
        pltpu.sync_copy(zer_hbm, ac[f])

    def start(c, b, sem):
        pltpu.async_copy(src_hbm.at[pl.ds(c * CBR, CBR)], sbuf[b], sem)
        pltpu.async_copy(dst_hbm.at[pl.ds(c * CBR, CBR)], dbuf[b], sem)

    def drain(b, sem):
        pltpu.make_async_copy(src_hbm.at[pl.ds(0, CBR)], sbuf[b], sem).wait()
        pltpu.make_async_copy(dst_hbm.at[pl.ds(0, CBR)], dbuf[b], sem).wait()

    def process(b):
        def row(r, carry2):
            for h in range(2):
                ks = range(h * 4, h * 4 + 4)
                s = [sbuf[b][r, pl.ds(k * 16, 16)] for k in ks]
                d = [dbuf[b][r, pl.ds(k * 16, 16)] for k in ks]
                g = [[plsc.load_gather(ys[f], [s[k]]) for f in range(FPW)]
                     for k in range(4)]
                for k in range(4):
                    for f in range(FPW):
                        plsc.addupdate_scatter(ac[f], [d[k]], g[k][f])
            return carry2

        lax.fori_loop(0, CBR, row, 0)

    start(0, 0, semA)

    def pair(i, carry):
        c0 = 2 * i
        drain(0, semA)
        start(c0 + 1, 1, semB)
        process(0)
        drain(1, semB)

        @pl.when(c0 + 2 < NCH)
        def _():
            start(c0 + 2, 0, semA)

        process(1)
        return carry

    lax.fori_loop(0, NCH // 2, pair, 0)
    for f in range(FPW):
        pltpu.sync_copy(ac[f], out_hbm.at[f0 + f])


def _prep_body(x_ref, w_ref, degp_ref, yt_ref, dinv_ref):
    deg = jnp.sum(degp_ref[...], axis=0, keepdims=True) + 1.0
    dinv = lax.rsqrt(deg)
    h = jnp.dot(x_ref[...], w_ref[...], preferred_element_type=jnp.float32)
    yt_ref[...] = h.T * dinv
    dinv_ref[...] = jnp.broadcast_to(dinv, (8, BLK))


def _tc_prep(xp, w, degp):
    return pl.pallas_call(
        _prep_body,
        grid=(NP // BLK,),
        in_specs=[
            pl.BlockSpec((BLK, D), lambda i: (i, 0)),
            pl.BlockSpec((D, D), lambda i: (0, 0)),
            pl.BlockSpec((NW, BLK), lambda i: (0, i)),
        ],
        out_specs=[
            pl.BlockSpec((D, BLK), lambda i: (0, i)),
            pl.BlockSpec((8, BLK), lambda i: (0, i)),
        ],
        out_shape=[
            jax.ShapeDtypeStruct((D, NP), jnp.float32),
            jax.ShapeDtypeStruct((8, NP), jnp.float32),
        ],
    )(xp, w, degp)


def _mid_body(p_ref, y0_ref, dinv_ref, b_ref, w_ref, y1_ref):
    dinv = dinv_ref[0:1, :]
    agg = p_ref[...] + y0_ref[...]
    h = jnp.maximum(agg * dinv + b_ref[...], 0.0)
    y1 = lax.dot_general(w_ref[...], h, (((0,), (0,)), ((), ())),
                         preferred_element_type=jnp.float32)
    y1_ref[...] = y1 * dinv


def _tc_mid(p, y0t, dinv8, bc, w):
    return pl.pallas_call(
        _mid_body,
        grid=(NP // BLK,),
        in_specs=[
            pl.BlockSpec((D, BLK), lambda i: (0, i)),
            pl.BlockSpec((D, BLK), lambda i: (0, i)),
            pl.BlockSpec((8, BLK), lambda i: (0, i)),
            pl.BlockSpec((D, 1), lambda i: (0, 0)),
            pl.BlockSpec((D, D), lambda i: (0, 0)),
        ],
        out_specs=pl.BlockSpec((D, BLK), lambda i: (0, i)),
        out_shape=jax.ShapeDtypeStruct((D, NP), jnp.float32),
    )(p, y0t, dinv8, bc, w)


def _fin_body(q_ref, y1_ref, dinv_ref, b_ref, z_ref):
    dinv = dinv_ref[0:1, :]
    zt = (q_ref[...] + y1_ref[...]) * dinv + b_ref[...]
    z_ref[...] = zt.T


def _tc_fin(q, y1t, dinv8, bc):
    return pl.pallas_call(
        _fin_body,
        grid=(NP // BLK,),
        in_specs=[
            pl.BlockSpec((D, BLK), lambda i: (0, i)),
            pl.BlockSpec((D, BLK), lambda i: (0, i)),
            pl.BlockSpec((8, BLK), lambda i: (0, i)),
            pl.BlockSpec((D, 1), lambda i: (0, 0)),
        ],
        out_specs=pl.BlockSpec((BLK, D), lambda i: (i, 0)),
        out_shape=jax.ShapeDtypeStruct((NP, D), jnp.float32),
    )(q, y1t, dinv8, bc)


def kernel(x1, edge_index1, x2, edge_index2, W0, b0, W1, b1):
    zer = jnp.zeros((NP,), jnp.float32)
    b0c = b0.reshape(D, 1)
    b1c = b1.reshape(D, 1)
    x1p = jnp.pad(x1, ((0, NP - N), (0, 0)))
    x2p = jnp.pad(x2, ((0, NP - N), (0, 0)))

    def prep_edges(ei):
        pad = jnp.full((EP - E,), N, dtype=jnp.int32)
        src = jnp.concatenate([ei[0].astype(jnp.int32), pad]).reshape(EBP, CH)
        dstf = jnp.concatenate([ei[1].astype(jnp.int32), pad])
        return src, dstf.reshape(EBP, CH), dstf.reshape(NW, CPW, CH)

    src1, dst1, dstp1 = prep_edges(edge_index1)
    src2, dst2, dstp2 = prep_edges(edge_index2)

    degp1 = _sc_degrees(dstp1, zer)
    degp2 = _sc_degrees(dstp2, zer)

    def backbone(xp, degp, src, dst):
        y0t, dinv8 = _tc_prep(xp, W0, degp)
        p = _sc_scatter(y0t, src, dst, zer)
        y1t = _tc_mid(p, y0t, dinv8, b0c, W1)
        q = _sc_scatter(y1t, src, dst, zer)
        return _tc_fin(q, y1t, dinv8, b1c)

    z1 = backbone(x1p, degp1, src1, dst1)[:N]
    z2 = backbone(x2p, degp2, src2, dst2)[:N]
    return (z1, z2)
